# baseline (speedup 1.0000x reference)
import functools

import jax
import jax.numpy as jnp
from jax import lax
from jax.experimental import pallas as pl
from jax.experimental.pallas import tpu as pltpu
from jax.experimental.pallas import tpu_sc as plsc

_VOCAB = 100000
_HIDDEN = 768
_MAX_POS = 2048
_BATCH = 4
_SEQ = 2048

_NSPLIT = 2
_BSPLIT = _BATCH // _NSPLIT
_NROW = _BSPLIT * _SEQ
_NW = 32
_R_PER_W = _NROW // _NW
_GCH = 64
_NGCH = _R_PER_W // _GCH
_EPS = 1e-12

_LN_ROWS = 256
_LN_GRID = _SEQ // _LN_ROWS


def _make_gather_body(b0):
    def body(x_hbm, tok_hbm, out_hbm, idx_v, buf0, buf1,
             gsem0, gsem1, ssem0, ssem1):
        nc = 2
        wid = lax.axis_index("s") * nc + lax.axis_index("c")
        r0 = wid * _R_PER_W
        b = b0 + wid // (_NW // _BSPLIT)
        s0 = (wid % (_NW // _BSPLIT)) * _R_PER_W
        pltpu.sync_copy(x_hbm.at[b, pl.ds(s0, _R_PER_W)], idx_v)

        bufs = (buf0, buf1)
        gsems = (gsem0, gsem1)
        ssems = (ssem0, ssem1)

        def gather(j):
            p = j % 2
            return pltpu.async_copy(
                tok_hbm.at[idx_v.at[pl.ds(j * _GCH, _GCH)]], bufs[p],
                gsems[p])

        def store(j):
            p = j % 2
            return pltpu.async_copy(
                bufs[p], out_hbm.at[pl.ds(r0 + j * _GCH, _GCH)], ssems[p])

        gh = {0: gather(0), 1: gather(1)}
        sh = {}
        for j in range(_NGCH):
            gh[j].wait()
            sh[j] = store(j)
            if j >= 1 and j + 1 < _NGCH:
                sh[j - 1].wait()
                gh[j + 1] = gather(j + 1)
        sh[_NGCH - 2].wait()
        sh[_NGCH - 1].wait()

    return body


def _sc_gather(x, token_table, b0):
    mesh = plsc.VectorSubcoreMesh(core_axis_name="c", subcore_axis_name="s")
    run = functools.partial(
        pl.kernel,
        mesh=mesh,
        out_type=jax.ShapeDtypeStruct((_NROW, _HIDDEN), jnp.float32),
        scratch_types=[
            pltpu.VMEM((_R_PER_W,), jnp.int32),
            pltpu.VMEM((_GCH, _HIDDEN), jnp.float32),
            pltpu.VMEM((_GCH, _HIDDEN), jnp.float32),
            pltpu.SemaphoreType.DMA,
            pltpu.SemaphoreType.DMA,
            pltpu.SemaphoreType.DMA,
            pltpu.SemaphoreType.DMA,
        ],
    )(_make_gather_body(b0))
    return run(x, token_table)


def _ln_body(tok_ref, pos_ref, gamma_ref, beta_ref, out_ref):
    h = tok_ref[...] + pos_ref[...][None]
    mean = jnp.mean(h, axis=-1, keepdims=True)
    d = h - mean
    var = jnp.mean(d * d, axis=-1, keepdims=True)
    out_ref[...] = d * lax.rsqrt(var + _EPS) * gamma_ref[...] \
        + beta_ref[...]


def _tc_layernorm(tok, pos_table, gamma, beta):
    return pl.pallas_call(
        _ln_body,
        grid=(_LN_GRID,),
        in_specs=[
            pl.BlockSpec((_BSPLIT, _LN_ROWS, _HIDDEN), lambda i: (0, i, 0)),
            pl.BlockSpec((_LN_ROWS, _HIDDEN), lambda i: (i, 0)),
            pl.BlockSpec((_HIDDEN,), lambda i: (0,)),
            pl.BlockSpec((_HIDDEN,), lambda i: (0,)),
        ],
        out_specs=pl.BlockSpec((_BSPLIT, _LN_ROWS, _HIDDEN),
                               lambda i: (0, i, 0)),
        out_shape=jax.ShapeDtypeStruct((_BSPLIT, _SEQ, _HIDDEN),
                                       jnp.float32),
    )(tok, pos_table, gamma, beta)


@jax.jit
def kernel(x, token_table, pos_table, gamma, beta):
    toks = [_sc_gather(x, token_table, h * _BSPLIT)
            for h in range(_NSPLIT)]
    outs = [_tc_layernorm(t.reshape(_BSPLIT, _SEQ, _HIDDEN),
                          pos_table, gamma, beta)
            for t in toks]
    return jnp.concatenate(outs, axis=0)


# speedup vs baseline: 1.2980x; 1.2980x over previous
import functools

import jax
import jax.numpy as jnp
from jax import lax
from jax.experimental import pallas as pl
from jax.experimental.pallas import tpu as pltpu
from jax.experimental.pallas import tpu_sc as plsc

_VOCAB = 100000
_HIDDEN = 768
_MAX_POS = 2048
_BATCH = 4
_SEQ = 2048

_NSPLIT = 1
_BSPLIT = _BATCH // _NSPLIT
_NROW = _BSPLIT * _SEQ
_NW = 32
_R_PER_W = _NROW // _NW
_GCH = 32
_NGCH = _R_PER_W // _GCH
_GBUF = 4
_EPS = 1e-12

_LN_ROWS = 128
_LN_GRID = _SEQ // _LN_ROWS


def _make_gather_body(b0):
    def body(x_hbm, tok_hbm, out_hbm, idx_v, buf0, buf1, buf2, buf3,
             gsem0, gsem1, gsem2, gsem3, ssem0, ssem1, ssem2, ssem3):
        nc = 2
        wid = lax.axis_index("s") * nc + lax.axis_index("c")
        r0 = wid * _R_PER_W
        b = b0 + wid // (_NW // _BSPLIT)
        s0 = (wid % (_NW // _BSPLIT)) * _R_PER_W
        pltpu.sync_copy(x_hbm.at[b, pl.ds(s0, _R_PER_W)], idx_v)

        bufs = (buf0, buf1, buf2, buf3)
        gsems = (gsem0, gsem1, gsem2, gsem3)
        ssems = (ssem0, ssem1, ssem2, ssem3)

        def gather(j):
            p = j % _GBUF
            return pltpu.async_copy(
                tok_hbm.at[idx_v.at[pl.ds(j * _GCH, _GCH)]], bufs[p],
                gsems[p])

        def store(j):
            p = j % _GBUF
            return pltpu.async_copy(
                bufs[p], out_hbm.at[pl.ds(r0 + j * _GCH, _GCH)], ssems[p])

        gh = {j: gather(j) for j in range(min(_GBUF - 1, _NGCH))}
        sh = {}
        waited = set()
        for j in range(_NGCH):
            gh[j].wait()
            sh[j] = store(j)
            nxt = j + _GBUF - 1
            if nxt < _NGCH:
                if j >= 1:
                    sh[j - 1].wait()
                    waited.add(j - 1)
                gh[nxt] = gather(nxt)
        for j in range(_NGCH):
            if j not in waited:
                sh[j].wait()

    return body


def _sc_gather(x, token_table, b0):
    mesh = plsc.VectorSubcoreMesh(core_axis_name="c", subcore_axis_name="s")
    run = functools.partial(
        pl.kernel,
        mesh=mesh,
        out_type=jax.ShapeDtypeStruct((_NROW, _HIDDEN), jnp.float32),
        scratch_types=(
            [pltpu.VMEM((_R_PER_W,), jnp.int32)]
            + [pltpu.VMEM((_GCH, _HIDDEN), jnp.float32)] * _GBUF
            + [pltpu.SemaphoreType.DMA] * (2 * _GBUF)
        ),
    )(_make_gather_body(b0))
    return run(x, token_table)


def _ln_body(tok_ref, pos_ref, gamma_ref, beta_ref, out_ref):
    h = tok_ref[...] + pos_ref[...][None]
    mean = jnp.mean(h, axis=-1, keepdims=True)
    d = h - mean
    var = jnp.mean(d * d, axis=-1, keepdims=True)
    out_ref[...] = d * lax.rsqrt(var + _EPS) * gamma_ref[...] \
        + beta_ref[...]


def _tc_layernorm(tok, pos_table, gamma, beta):
    return pl.pallas_call(
        _ln_body,
        grid=(_LN_GRID,),
        in_specs=[
            pl.BlockSpec((_BSPLIT, _LN_ROWS, _HIDDEN), lambda i: (0, i, 0)),
            pl.BlockSpec((_LN_ROWS, _HIDDEN), lambda i: (i, 0)),
            pl.BlockSpec((_HIDDEN,), lambda i: (0,)),
            pl.BlockSpec((_HIDDEN,), lambda i: (0,)),
        ],
        out_specs=pl.BlockSpec((_BSPLIT, _LN_ROWS, _HIDDEN),
                               lambda i: (0, i, 0)),
        out_shape=jax.ShapeDtypeStruct((_BSPLIT, _SEQ, _HIDDEN),
                                       jnp.float32),
    )(tok, pos_table, gamma, beta)


@jax.jit
def kernel(x, token_table, pos_table, gamma, beta):
    toks = [_sc_gather(x, token_table, h * _BSPLIT)
            for h in range(_NSPLIT)]
    outs = [_tc_layernorm(t.reshape(_BSPLIT, _SEQ, _HIDDEN),
                          pos_table, gamma, beta)
            for t in toks]
    return jnp.concatenate(outs, axis=0)


# speedup vs baseline: 1.3949x; 1.0747x over previous
import functools

import jax
import jax.numpy as jnp
from jax import lax
from jax.experimental import pallas as pl
from jax.experimental.pallas import tpu as pltpu
from jax.experimental.pallas import tpu_sc as plsc

_VOCAB = 100000
_HIDDEN = 768
_MAX_POS = 2048
_BATCH = 4
_SEQ = 2048

_NSPLIT = 1
_BSPLIT = _BATCH // _NSPLIT
_NROW = _BSPLIT * _SEQ
_NW = 32
_R_PER_W = _NROW // _NW
_GCH = 32
_NGCH = _R_PER_W // _GCH
_GBUF = 4
_EPS = 1e-12

_LN_ROWS = 256
_LN_GRID = _SEQ // _LN_ROWS


def _make_gather_body(b0):
    def body(x_hbm, tok_hbm, out_hbm, idx_v, buf0, buf1, buf2, buf3,
             gsem0, gsem1, gsem2, gsem3, ssem0, ssem1, ssem2, ssem3):
        nc = 2
        wid = lax.axis_index("s") * nc + lax.axis_index("c")
        r0 = wid * _R_PER_W
        b = b0 + wid // (_NW // _BSPLIT)
        s0 = (wid % (_NW // _BSPLIT)) * _R_PER_W
        pltpu.sync_copy(x_hbm.at[b, pl.ds(s0, _R_PER_W)], idx_v)

        bufs = (buf0, buf1, buf2, buf3)
        gsems = (gsem0, gsem1, gsem2, gsem3)
        ssems = (ssem0, ssem1, ssem2, ssem3)

        def gather(j):
            p = j % _GBUF
            return pltpu.async_copy(
                tok_hbm.at[idx_v.at[pl.ds(j * _GCH, _GCH)]], bufs[p],
                gsems[p])

        def store(j):
            p = j % _GBUF
            return pltpu.async_copy(
                bufs[p], out_hbm.at[pl.ds(r0 + j * _GCH, _GCH)], ssems[p])

        gh = {j: gather(j) for j in range(min(_GBUF - 1, _NGCH))}
        sh = {}
        waited = set()
        for j in range(_NGCH):
            gh[j].wait()
            sh[j] = store(j)
            nxt = j + _GBUF - 1
            if nxt < _NGCH:
                if j >= 1:
                    sh[j - 1].wait()
                    waited.add(j - 1)
                gh[nxt] = gather(nxt)
        for j in range(_NGCH):
            if j not in waited:
                sh[j].wait()

    return body


def _sc_gather(x, token_table, b0):
    mesh = plsc.VectorSubcoreMesh(core_axis_name="c", subcore_axis_name="s")
    run = functools.partial(
        pl.kernel,
        mesh=mesh,
        out_type=jax.ShapeDtypeStruct((_NROW, _HIDDEN), jnp.float32),
        scratch_types=(
            [pltpu.VMEM((_R_PER_W,), jnp.int32)]
            + [pltpu.VMEM((_GCH, _HIDDEN), jnp.float32)] * _GBUF
            + [pltpu.SemaphoreType.DMA] * (2 * _GBUF)
        ),
    )(_make_gather_body(b0))
    return run(x, token_table)


def _ln_body(tok_ref, pos_ref, gamma_ref, beta_ref, out_ref):
    h = tok_ref[...] + pos_ref[...][None]
    mean = jnp.mean(h, axis=-1, keepdims=True)
    d = h - mean
    var = jnp.mean(d * d, axis=-1, keepdims=True)
    out_ref[...] = d * lax.rsqrt(var + _EPS) * gamma_ref[...] \
        + beta_ref[...]


def _tc_layernorm(tok, pos_table, gamma, beta):
    return pl.pallas_call(
        _ln_body,
        grid=(_LN_GRID,),
        in_specs=[
            pl.BlockSpec((_BSPLIT, _LN_ROWS, _HIDDEN), lambda i: (0, i, 0)),
            pl.BlockSpec((_LN_ROWS, _HIDDEN), lambda i: (i, 0)),
            pl.BlockSpec((_HIDDEN,), lambda i: (0,)),
            pl.BlockSpec((_HIDDEN,), lambda i: (0,)),
        ],
        out_specs=pl.BlockSpec((_BSPLIT, _LN_ROWS, _HIDDEN),
                               lambda i: (0, i, 0)),
        out_shape=jax.ShapeDtypeStruct((_BSPLIT, _SEQ, _HIDDEN),
                                       jnp.float32),
    )(tok, pos_table, gamma, beta)


@jax.jit
def kernel(x, token_table, pos_table, gamma, beta):
    toks = [_sc_gather(x, token_table, h * _BSPLIT)
            for h in range(_NSPLIT)]
    outs = [_tc_layernorm(t.reshape(_BSPLIT, _SEQ, _HIDDEN),
                          pos_table, gamma, beta)
            for t in toks]
    return jnp.concatenate(outs, axis=0)


# speedup vs baseline: 1.4008x; 1.0042x over previous
import functools

import jax
import jax.numpy as jnp
from jax import lax
from jax.experimental import pallas as pl
from jax.experimental.pallas import tpu as pltpu
from jax.experimental.pallas import tpu_sc as plsc

_VOCAB = 100000
_HIDDEN = 768
_MAX_POS = 2048
_BATCH = 4
_SEQ = 2048

_NSPLIT = 1
_BSPLIT = _BATCH // _NSPLIT
_NROW = _BSPLIT * _SEQ
_NW = 32
_R_PER_W = _NROW // _NW
_GCH = 32
_NGCH = _R_PER_W // _GCH
_GBUF = 4
_EPS = 1e-12

_LN_ROWS = 512
_LN_GRID = _SEQ // _LN_ROWS


def _make_gather_body(b0):
    def body(x_hbm, tok_hbm, out_hbm, idx_v, buf0, buf1, buf2, buf3,
             gsem0, gsem1, gsem2, gsem3, ssem0, ssem1, ssem2, ssem3):
        nc = 2
        wid = lax.axis_index("s") * nc + lax.axis_index("c")
        r0 = wid * _R_PER_W
        b = b0 + wid // (_NW // _BSPLIT)
        s0 = (wid % (_NW // _BSPLIT)) * _R_PER_W
        pltpu.sync_copy(x_hbm.at[b, pl.ds(s0, _R_PER_W)], idx_v)

        bufs = (buf0, buf1, buf2, buf3)
        gsems = (gsem0, gsem1, gsem2, gsem3)
        ssems = (ssem0, ssem1, ssem2, ssem3)

        def gather(j):
            p = j % _GBUF
            return pltpu.async_copy(
                tok_hbm.at[idx_v.at[pl.ds(j * _GCH, _GCH)]], bufs[p],
                gsems[p])

        def store(j):
            p = j % _GBUF
            return pltpu.async_copy(
                bufs[p], out_hbm.at[pl.ds(r0 + j * _GCH, _GCH)], ssems[p])

        gh = {j: gather(j) for j in range(min(_GBUF - 1, _NGCH))}
        sh = {}
        waited = set()
        for j in range(_NGCH):
            gh[j].wait()
            sh[j] = store(j)
            nxt = j + _GBUF - 1
            if nxt < _NGCH:
                if j >= 1:
                    sh[j - 1].wait()
                    waited.add(j - 1)
                gh[nxt] = gather(nxt)
        for j in range(_NGCH):
            if j not in waited:
                sh[j].wait()

    return body


def _sc_gather(x, token_table, b0):
    mesh = plsc.VectorSubcoreMesh(core_axis_name="c", subcore_axis_name="s")
    run = functools.partial(
        pl.kernel,
        mesh=mesh,
        out_type=jax.ShapeDtypeStruct((_NROW, _HIDDEN), jnp.float32),
        scratch_types=(
            [pltpu.VMEM((_R_PER_W,), jnp.int32)]
            + [pltpu.VMEM((_GCH, _HIDDEN), jnp.float32)] * _GBUF
            + [pltpu.SemaphoreType.DMA] * (2 * _GBUF)
        ),
    )(_make_gather_body(b0))
    return run(x, token_table)


def _ln_body(tok_ref, pos_ref, gamma_ref, beta_ref, out_ref):
    h = tok_ref[...] + pos_ref[...][None]
    mean = jnp.mean(h, axis=-1, keepdims=True)
    d = h - mean
    var = jnp.mean(d * d, axis=-1, keepdims=True)
    out_ref[...] = d * lax.rsqrt(var + _EPS) * gamma_ref[...] \
        + beta_ref[...]


def _tc_layernorm(tok, pos_table, gamma, beta):
    return pl.pallas_call(
        _ln_body,
        grid=(_LN_GRID,),
        in_specs=[
            pl.BlockSpec((_BSPLIT, _LN_ROWS, _HIDDEN), lambda i: (0, i, 0)),
            pl.BlockSpec((_LN_ROWS, _HIDDEN), lambda i: (i, 0)),
            pl.BlockSpec((_HIDDEN,), lambda i: (0,)),
            pl.BlockSpec((_HIDDEN,), lambda i: (0,)),
        ],
        out_specs=pl.BlockSpec((_BSPLIT, _LN_ROWS, _HIDDEN),
                               lambda i: (0, i, 0)),
        out_shape=jax.ShapeDtypeStruct((_BSPLIT, _SEQ, _HIDDEN),
                                       jnp.float32),
    )(tok, pos_table, gamma, beta)


@jax.jit
def kernel(x, token_table, pos_table, gamma, beta):
    toks = [_sc_gather(x, token_table, h * _BSPLIT)
            for h in range(_NSPLIT)]
    outs = [_tc_layernorm(t.reshape(_BSPLIT, _SEQ, _HIDDEN),
                          pos_table, gamma, beta)
            for t in toks]
    return jnp.concatenate(outs, axis=0)
